# Initial kernel scaffold; baseline (speedup 1.0000x reference)
import jax
import jax.numpy as jnp
from jax.experimental import pallas as pl


def kernel(x, y):
    raise NotImplementedError("write your pallas kernel here")


# speedup vs baseline: 743.7522x; 743.7522x over previous
import functools

import jax
import jax.numpy as jnp
from jax import lax
from jax.experimental import pallas as pl
from jax.experimental.pallas import tpu as pltpu
from jax.experimental.pallas import tpu_sc as plsc

N_KNOTS = 21
X_MIN = -5.0
X_MAX = 5.0
IN_FEATURES = 1024
DX = (X_MAX - X_MIN) / (N_KNOTS - 1)
INV_DX = 1.0 / DX

NC = 2
NS = 16
NW = NC * NS
LANES = 16

ROWS_PER_CHUNK = 8
CHUNK = ROWS_PER_CHUNK * IN_FEATURES


def _spline_body(x_hbm, y_hbm, out_hbm, y_v, x_v, o_v, *, n_chunks):
    wid = lax.axis_index("s") * NC + lax.axis_index("c")
    base = wid * (n_chunks * CHUNK)

    pltpu.sync_copy(y_hbm, y_v)

    iota = lax.iota(jnp.int32, LANES)

    def chunk_body(ci, carry):
        off = base + ci * CHUNK
        pltpu.sync_copy(x_hbm.at[pl.ds(off, CHUNK)], x_v)

        def row_body(r, carry2):
            rb = r * IN_FEATURES
            for j in range(IN_FEATURES // LANES):
                sl = pl.ds(rb + j * LANES, LANES)
                xv = x_v[sl]
                xc = jnp.minimum(jnp.maximum(xv, X_MIN), X_MAX)
                pos = (xc - X_MIN) * INV_DX
                idx = jnp.minimum(pos.astype(jnp.int32), N_KNOTS - 2)
                t = pos - idx.astype(jnp.float32)
                flat = iota * N_KNOTS + (j * LANES * N_KNOTS) + idx
                y_l = plsc.load_gather(y_v, [flat])
                y_r = plsc.load_gather(y_v, [flat + 1])
                o_v[sl] = y_l + t * (y_r - y_l)
            return carry2

        lax.fori_loop(0, ROWS_PER_CHUNK, row_body, 0)
        pltpu.sync_copy(o_v, out_hbm.at[pl.ds(off, CHUNK)])
        return carry

    lax.fori_loop(0, n_chunks, chunk_body, 0)


def kernel(x, y):
    orig_shape = x.shape
    n = x.size
    assert n % (NW * CHUNK) == 0
    n_chunks = n // (NW * CHUNK)

    x_flat = x.reshape(n)
    mesh = plsc.VectorSubcoreMesh(core_axis_name="c", subcore_axis_name="s")
    run = pl.kernel(
        functools.partial(_spline_body, n_chunks=n_chunks),
        out_type=jax.ShapeDtypeStruct((n,), jnp.float32),
        mesh=mesh,
        compiler_params=pltpu.CompilerParams(needs_layout_passes=False),
        scratch_types=[
            pltpu.VMEM((IN_FEATURES * N_KNOTS,), jnp.float32),
            pltpu.VMEM((CHUNK,), jnp.float32),
            pltpu.VMEM((CHUNK,), jnp.float32),
        ],
    )
    out_flat = run(x_flat, y.reshape(IN_FEATURES * N_KNOTS))
    return out_flat.reshape(orig_shape)
